# baseline (speedup 1.0000x reference)
import functools

import jax
import jax.numpy as jnp
from jax import lax
from jax.experimental import pallas as pl
from jax.experimental.pallas import tpu as pltpu, tpu_sc as plsc

N = 10000
E = 320000
D = 128
NC = 2
NS = 16
NW = NC * NS
CH = 40
EPT = 10240
EPAD = NW * EPT - E
MB = 8
MBC = EPT // CH // MB
NB = 8
STG = 4
PPT = EPT - E // NW
NDUMP = 128
NA = N + NDUMP
RPT = 632
TAIL = N - (NS - 1) * RPT
BM = 1000
GRID = N // BM

_mesh = plsc.VectorSubcoreMesh(core_axis_name="c", subcore_axis_name="s")


def _rows_copy(sid, fn):
    r0 = pl.multiple_of(sid * RPT, 8)

    @pl.when(sid < NS - 1)
    def _():
        fn(r0, RPT)

    @pl.when(sid == NS - 1)
    def _():
        fn((NS - 1) * RPT, TAIL)


@functools.partial(
    pl.kernel, mesh=_mesh,
    out_type=jax.ShapeDtypeStruct((NC, NA, D), jnp.float32),
    scratch_types=[
        pltpu.VMEM_SHARED((NA, D), jnp.float32),
        pltpu.VMEM((CH, D), jnp.float32),
        pltpu.VMEM((MBC, CH), jnp.int32),
    ] + [pltpu.SemaphoreType.DMA] * NB,
)
def _sc_degree(col_hbm, ones_hbm, zeros_hbm, deg_hbm, acc_sh, ones_v,
               cidx_v, *sems):
    cid = lax.axis_index("c")
    sid = lax.axis_index("s")
    wid = sid * NC + cid
    _rows_copy(sid, lambda b, n: pltpu.sync_copy(
        zeros_hbm.at[pl.ds(b, n)], acc_sh.at[pl.ds(b, n)]))
    pltpu.sync_copy(ones_hbm, ones_v)
    plsc.subcore_barrier()

    def body(m, carry):
        pltpu.sync_copy(col_hbm.at[wid, m], cidx_v)
        for blk in range(MBC // NB):
            sd = [
                pltpu.async_copy(ones_v, acc_sh.at[cidx_v.at[blk * NB + b]],
                                 sems[b], add=True)
                for b in range(NB)
            ]
            for d in sd:
                d.wait()
        return carry

    lax.fori_loop(0, MB, body, 0)
    plsc.subcore_barrier()
    _rows_copy(sid, lambda b, n: pltpu.sync_copy(
        acc_sh.at[pl.ds(b, n)], deg_hbm.at[cid, pl.ds(b, n)]))


@functools.partial(
    pl.kernel, mesh=_mesh,
    out_type=jax.ShapeDtypeStruct((NC, NA, D), jnp.float32),
    scratch_types=[
        pltpu.VMEM_SHARED((NA, D), jnp.float32),
    ] + [pltpu.VMEM((CH, D), jnp.float32)] * NB + [
        pltpu.VMEM((MBC, CH), jnp.int32),
        pltpu.VMEM((MBC, CH), jnp.int32),
    ] + [pltpu.SemaphoreType.DMA] * (2 * NB),
)
def _sc_scatter(y_hbm, row_hbm, col_hbm, zeros_hbm, acc_hbm, acc_sh, *rest):
    rows = rest[:NB]
    ridx_v = rest[NB]
    cidx_v = rest[NB + 1]
    sem_g = rest[NB + 2:NB + 2 + NB]
    sem_s = rest[NB + 2 + NB:]
    cid = lax.axis_index("c")
    sid = lax.axis_index("s")
    wid = sid * NC + cid
    _rows_copy(sid, lambda b, n: pltpu.sync_copy(
        zeros_hbm.at[pl.ds(b, n)], acc_sh.at[pl.ds(b, n)]))
    plsc.subcore_barrier()

    def body(m, carry):
        pltpu.sync_copy(row_hbm.at[wid, m], ridx_v)
        pltpu.sync_copy(col_hbm.at[wid, m], cidx_v)
        gd = [None] * MBC
        sd = [None] * MBC
        for k in range(MBC):
            b = k % NB
            if k >= NB:
                sd[k - NB].wait()
            gd[k] = pltpu.async_copy(
                y_hbm.at[ridx_v.at[k]], rows[b], sem_g[b])
            if k >= STG:
                j = k - STG
                gd[j].wait()
                sd[j] = pltpu.async_copy(
                    rows[j % NB], acc_sh.at[cidx_v.at[j]],
                    sem_s[j % NB], add=True)
        for j in range(MBC - STG, MBC):
            gd[j].wait()
            sd[j] = pltpu.async_copy(
                rows[j % NB], acc_sh.at[cidx_v.at[j]], sem_s[j % NB], add=True)
        for j in range(MBC - NB, MBC):
            sd[j].wait()
        return carry

    lax.fori_loop(0, MB, body, 0)
    plsc.subcore_barrier()
    _rows_copy(sid, lambda b, n: pltpu.sync_copy(
        acc_sh.at[pl.ds(b, n)], acc_hbm.at[cid, pl.ds(b, n)]))


def _dinv_from_deg(dp):
    deg = dp[0, :, 0] + dp[1, :, 0] + 1.0
    return lax.rsqrt(deg)[:, None]


def _tc1_body(dp_ref, x_ref, w_ref, y_ref):
    dinv = _dinv_from_deg(dp_ref[...])
    xw = jnp.dot(x_ref[...], w_ref[...], preferred_element_type=jnp.float32)
    y_ref[...] = xw * dinv


_tc1 = pl.pallas_call(
    _tc1_body,
    grid=(GRID,),
    in_specs=[
        pl.BlockSpec((NC, BM, D), lambda i: (0, i, 0)),
        pl.BlockSpec((BM, D), lambda i: (i, 0)),
        pl.BlockSpec((D, D), lambda i: (0, 0)),
    ],
    out_specs=pl.BlockSpec((BM, D), lambda i: (i, 0)),
    out_shape=jax.ShapeDtypeStruct((N, D), jnp.float32),
)


def _tc2_body(dp_ref, a_ref, y1_ref, b_ref, w_ref, y2_ref):
    dinv = _dinv_from_deg(dp_ref[...])
    a = a_ref[...]
    s = a[0] + a[1] + y1_ref[...]
    h = jnp.maximum(s * dinv + b_ref[...], 0.0)
    hw = jnp.dot(h, w_ref[...], preferred_element_type=jnp.float32)
    y2_ref[...] = hw * dinv


_tc2 = pl.pallas_call(
    _tc2_body,
    grid=(GRID,),
    in_specs=[
        pl.BlockSpec((NC, BM, D), lambda i: (0, i, 0)),
        pl.BlockSpec((NC, BM, D), lambda i: (0, i, 0)),
        pl.BlockSpec((BM, D), lambda i: (i, 0)),
        pl.BlockSpec((1, D), lambda i: (0, 0)),
        pl.BlockSpec((D, D), lambda i: (0, 0)),
    ],
    out_specs=pl.BlockSpec((BM, D), lambda i: (i, 0)),
    out_shape=jax.ShapeDtypeStruct((N, D), jnp.float32),
)


def _tc3_body(dp_ref, a_ref, y2_ref, b_ref, o_ref):
    dinv = _dinv_from_deg(dp_ref[...])
    a = a_ref[...]
    o_ref[...] = (a[0] + a[1] + y2_ref[...]) * dinv + b_ref[...]


_tc3 = pl.pallas_call(
    _tc3_body,
    grid=(GRID,),
    in_specs=[
        pl.BlockSpec((NC, BM, D), lambda i: (0, i, 0)),
        pl.BlockSpec((NC, BM, D), lambda i: (0, i, 0)),
        pl.BlockSpec((BM, D), lambda i: (i, 0)),
        pl.BlockSpec((1, D), lambda i: (0, 0)),
    ],
    out_specs=pl.BlockSpec((BM, D), lambda i: (i, 0)),
    out_shape=jax.ShapeDtypeStruct((N, D), jnp.float32),
)


def kernel(x, edge_index, W1, b1, W2, b2):
    pad_row = ((jnp.arange(NW * PPT, dtype=jnp.int32) * 131) % N).reshape(NW, PPT)
    pad_col = N + (jnp.arange(NW * PPT, dtype=jnp.int32) % NDUMP).reshape(NW, PPT)
    row = jnp.concatenate([edge_index[0].reshape(NW, -1), pad_row],
                          axis=1).reshape(NW, MB, MBC, CH)
    col = jnp.concatenate([edge_index[1].reshape(NW, -1), pad_col],
                          axis=1).reshape(NW, MB, MBC, CH)
    ones = jnp.ones((CH, D), jnp.float32)
    zeros = jnp.zeros((NA, D), jnp.float32)
    deg = _sc_degree(col, ones, zeros)
    y1 = _tc1(deg, x, W1)
    acc1 = _sc_scatter(y1, row, col, zeros)
    y2 = _tc2(deg, acc1, y1, b1.reshape(1, D), W2)
    acc2 = _sc_scatter(y2, row, col, zeros)
    out = _tc3(deg, acc2, y2, b2.reshape(1, D))
    return out


# speedup vs baseline: 1.1118x; 1.1118x over previous
import functools

import jax
import jax.numpy as jnp
from jax import lax
from jax.experimental import pallas as pl
from jax.experimental.pallas import tpu as pltpu, tpu_sc as plsc

N = 10000
E = 320000
D = 128
NC = 2
NS = 16
NW = NC * NS
CH = 80
EPT = 10240
EPAD = NW * EPT - E
MB = 8
MBC = EPT // CH // MB
NB = 4
STG = 2
PPT = EPT - E // NW
NDUMP = 128
NA = N + NDUMP
RPT = 632
TAIL = N - (NS - 1) * RPT
BM = 1000
GRID = N // BM

_mesh = plsc.VectorSubcoreMesh(core_axis_name="c", subcore_axis_name="s")


def _rows_copy(sid, fn):
    r0 = pl.multiple_of(sid * RPT, 8)

    @pl.when(sid < NS - 1)
    def _():
        fn(r0, RPT)

    @pl.when(sid == NS - 1)
    def _():
        fn((NS - 1) * RPT, TAIL)


@functools.partial(
    pl.kernel, mesh=_mesh,
    out_type=jax.ShapeDtypeStruct((NC, NA, D), jnp.float32),
    scratch_types=[
        pltpu.VMEM_SHARED((NA, D), jnp.float32),
        pltpu.VMEM((CH, D), jnp.float32),
        pltpu.VMEM((MBC, CH), jnp.int32),
        pltpu.VMEM((MBC, CH), jnp.int32),
        pltpu.SemaphoreType.DMA,
        pltpu.SemaphoreType.DMA,
    ] + [pltpu.SemaphoreType.DMA] * NB,
)
def _sc_degree(col_hbm, ones_hbm, zeros_hbm, deg_hbm, acc_sh, ones_v,
               cidx_a, cidx_b, sem_ia, sem_ib, *sems):
    cid = lax.axis_index("c")
    sid = lax.axis_index("s")
    wid = sid * NC + cid
    _rows_copy(sid, lambda b, n: pltpu.sync_copy(
        zeros_hbm.at[pl.ds(b, n)], acc_sh.at[pl.ds(b, n)]))
    pltpu.sync_copy(ones_hbm, ones_v)
    plsc.subcore_barrier()

    idx_bufs = (cidx_a, cidx_b)
    idx_sems = (sem_ia, sem_ib)
    idd = pltpu.async_copy(col_hbm.at[wid, 0], cidx_a, sem_ia)
    sd = [None] * NB
    for m in range(MB):
        cidx_v = idx_bufs[m % 2]
        idd.wait()
        for k in range(MBC):
            if k == NB and m + 1 < MB:
                idd = pltpu.async_copy(col_hbm.at[wid, m + 1],
                                       idx_bufs[(m + 1) % 2],
                                       idx_sems[(m + 1) % 2])
            b = (m * MBC + k) % NB
            if sd[b] is not None:
                sd[b].wait()
            sd[b] = pltpu.async_copy(ones_v, acc_sh.at[cidx_v.at[k]],
                                     sems[b], add=True)
    for d in sd:
        d.wait()
    plsc.subcore_barrier()
    _rows_copy(sid, lambda b, n: pltpu.sync_copy(
        acc_sh.at[pl.ds(b, n)], deg_hbm.at[cid, pl.ds(b, n)]))


@functools.partial(
    pl.kernel, mesh=_mesh,
    out_type=jax.ShapeDtypeStruct((NC, NA, D), jnp.float32),
    scratch_types=[
        pltpu.VMEM_SHARED((NA, D), jnp.float32),
    ] + [pltpu.VMEM((CH, D), jnp.float32)] * NB + [
        pltpu.VMEM((MBC, CH), jnp.int32),
        pltpu.VMEM((MBC, CH), jnp.int32),
        pltpu.VMEM((MBC, CH), jnp.int32),
        pltpu.VMEM((MBC, CH), jnp.int32),
        pltpu.SemaphoreType.DMA,
        pltpu.SemaphoreType.DMA,
    ] + [pltpu.SemaphoreType.DMA] * (2 * NB),
)
def _sc_scatter(y_hbm, row_hbm, col_hbm, zeros_hbm, acc_hbm, acc_sh, *rest):
    rows = rest[:NB]
    idx_bufs = ((rest[NB], rest[NB + 1]), (rest[NB + 2], rest[NB + 3]))
    idx_sems = (rest[NB + 4], rest[NB + 5])
    sem_g = rest[NB + 6:NB + 6 + NB]
    sem_s = rest[NB + 6 + NB:]
    cid = lax.axis_index("c")
    sid = lax.axis_index("s")
    wid = sid * NC + cid
    _rows_copy(sid, lambda b, n: pltpu.sync_copy(
        zeros_hbm.at[pl.ds(b, n)], acc_sh.at[pl.ds(b, n)]))
    plsc.subcore_barrier()

    NCH = MB * MBC

    def issue_idx(m):
        r, c = idx_bufs[m % 2]
        s = idx_sems[m % 2]
        return (pltpu.async_copy(row_hbm.at[wid, m], r, s),
                pltpu.async_copy(col_hbm.at[wid, m], c, s))

    def issue_scatter(j):
        jm, jk = divmod(j, MBC)
        cb = idx_bufs[jm % 2][1]
        return pltpu.async_copy(rows[j % NB], acc_sh.at[cb.at[jk]],
                                sem_s[j % NB], add=True)

    gd = [None] * NCH
    sd = [None] * NCH
    idd = issue_idx(0)
    for kk in range(NCH):
        m, k = divmod(kk, MBC)
        if k == 0:
            for d in idd:
                d.wait()
            ridx_v = idx_bufs[m % 2][0]
        if k == NB and m + 1 < MB:
            idd = issue_idx(m + 1)
        if kk >= NB:
            sd[kk - NB].wait()
        gd[kk] = pltpu.async_copy(y_hbm.at[ridx_v.at[k]], rows[kk % NB],
                                  sem_g[kk % NB])
        if kk >= STG:
            gd[kk - STG].wait()
            sd[kk - STG] = issue_scatter(kk - STG)
    for j in range(NCH - STG, NCH):
        gd[j].wait()
        sd[j] = issue_scatter(j)
    for j in range(NCH - NB, NCH):
        sd[j].wait()
    plsc.subcore_barrier()
    _rows_copy(sid, lambda b, n: pltpu.sync_copy(
        acc_sh.at[pl.ds(b, n)], acc_hbm.at[cid, pl.ds(b, n)]))


def _dinv_from_deg(dp):
    deg = dp[0, :, 0] + dp[1, :, 0] + 1.0
    return lax.rsqrt(deg)[:, None]


def _tc1_body(dp_ref, x_ref, w_ref, y_ref):
    dinv = _dinv_from_deg(dp_ref[...])
    xw = jnp.dot(x_ref[...], w_ref[...], preferred_element_type=jnp.float32)
    y_ref[...] = xw * dinv


_tc1 = pl.pallas_call(
    _tc1_body,
    grid=(GRID,),
    in_specs=[
        pl.BlockSpec((NC, BM, D), lambda i: (0, i, 0)),
        pl.BlockSpec((BM, D), lambda i: (i, 0)),
        pl.BlockSpec((D, D), lambda i: (0, 0)),
    ],
    out_specs=pl.BlockSpec((BM, D), lambda i: (i, 0)),
    out_shape=jax.ShapeDtypeStruct((N, D), jnp.float32),
)


def _tc2_body(dp_ref, a_ref, y1_ref, b_ref, w_ref, y2_ref):
    dinv = _dinv_from_deg(dp_ref[...])
    a = a_ref[...]
    s = a[0] + a[1] + y1_ref[...]
    h = jnp.maximum(s * dinv + b_ref[...], 0.0)
    hw = jnp.dot(h, w_ref[...], preferred_element_type=jnp.float32)
    y2_ref[...] = hw * dinv


_tc2 = pl.pallas_call(
    _tc2_body,
    grid=(GRID,),
    in_specs=[
        pl.BlockSpec((NC, BM, D), lambda i: (0, i, 0)),
        pl.BlockSpec((NC, BM, D), lambda i: (0, i, 0)),
        pl.BlockSpec((BM, D), lambda i: (i, 0)),
        pl.BlockSpec((1, D), lambda i: (0, 0)),
        pl.BlockSpec((D, D), lambda i: (0, 0)),
    ],
    out_specs=pl.BlockSpec((BM, D), lambda i: (i, 0)),
    out_shape=jax.ShapeDtypeStruct((N, D), jnp.float32),
)


def _tc3_body(dp_ref, a_ref, y2_ref, b_ref, o_ref):
    dinv = _dinv_from_deg(dp_ref[...])
    a = a_ref[...]
    o_ref[...] = (a[0] + a[1] + y2_ref[...]) * dinv + b_ref[...]


_tc3 = pl.pallas_call(
    _tc3_body,
    grid=(GRID,),
    in_specs=[
        pl.BlockSpec((NC, BM, D), lambda i: (0, i, 0)),
        pl.BlockSpec((NC, BM, D), lambda i: (0, i, 0)),
        pl.BlockSpec((BM, D), lambda i: (i, 0)),
        pl.BlockSpec((1, D), lambda i: (0, 0)),
    ],
    out_specs=pl.BlockSpec((BM, D), lambda i: (i, 0)),
    out_shape=jax.ShapeDtypeStruct((N, D), jnp.float32),
)


def kernel(x, edge_index, W1, b1, W2, b2):
    pad_row = ((jnp.arange(NW * PPT, dtype=jnp.int32) * 131) % N).reshape(NW, PPT)
    pad_col = N + (jnp.arange(NW * PPT, dtype=jnp.int32) % NDUMP).reshape(NW, PPT)
    row = jnp.concatenate([edge_index[0].reshape(NW, -1), pad_row],
                          axis=1).reshape(NW, MB, MBC, CH)
    col = jnp.concatenate([edge_index[1].reshape(NW, -1), pad_col],
                          axis=1).reshape(NW, MB, MBC, CH)
    ones = jnp.ones((CH, D), jnp.float32)
    zeros = jnp.zeros((NA, D), jnp.float32)
    deg = _sc_degree(col, ones, zeros)
    y1 = _tc1(deg, x, W1)
    acc1 = _sc_scatter(y1, row, col, zeros)
    y2 = _tc2(deg, acc1, y1, b1.reshape(1, D), W2)
    acc2 = _sc_scatter(y2, row, col, zeros)
    out = _tc3(deg, acc2, y2, b2.reshape(1, D))
    return out


# speedup vs baseline: 1.1343x; 1.0202x over previous
import functools

import jax
import jax.numpy as jnp
from jax import lax
from jax.experimental import pallas as pl
from jax.experimental.pallas import tpu as pltpu, tpu_sc as plsc

N = 10000
E = 320000
D = 128
NC = 2
NS = 16
NW = NC * NS
CH = 80
EPT = E // NW
MB = 25
MBC = EPT // CH // MB
NB = 4
STG = 2
NA = N
RPT = 632
TAIL = N - (NS - 1) * RPT
BM = 1000
GRID = N // BM

_mesh = plsc.VectorSubcoreMesh(core_axis_name="c", subcore_axis_name="s")


def _rows_copy(sid, fn):
    r0 = pl.multiple_of(sid * RPT, 8)

    @pl.when(sid < NS - 1)
    def _():
        fn(r0, RPT)

    @pl.when(sid == NS - 1)
    def _():
        fn((NS - 1) * RPT, TAIL)


@functools.partial(
    pl.kernel, mesh=_mesh,
    out_type=jax.ShapeDtypeStruct((NC, NA, D), jnp.float32),
    scratch_types=[
        pltpu.VMEM_SHARED((NA, D), jnp.float32),
        pltpu.VMEM((CH, D), jnp.float32),
        pltpu.VMEM((MBC, CH), jnp.int32),
        pltpu.VMEM((MBC, CH), jnp.int32),
        pltpu.SemaphoreType.DMA,
        pltpu.SemaphoreType.DMA,
    ] + [pltpu.SemaphoreType.DMA] * NB,
)
def _sc_degree(col_hbm, ones_hbm, zeros_hbm, deg_hbm, acc_sh, ones_v,
               cidx_a, cidx_b, sem_ia, sem_ib, *sems):
    cid = lax.axis_index("c")
    sid = lax.axis_index("s")
    wid = sid * NC + cid
    _rows_copy(sid, lambda b, n: pltpu.sync_copy(
        zeros_hbm.at[pl.ds(b, n)], acc_sh.at[pl.ds(b, n)]))
    pltpu.sync_copy(ones_hbm, ones_v)
    plsc.subcore_barrier()

    idx_bufs = (cidx_a, cidx_b)
    idx_sems = (sem_ia, sem_ib)
    idd = pltpu.async_copy(col_hbm.at[wid, 0], cidx_a, sem_ia)
    sd = [None] * NB
    for m in range(MB):
        cidx_v = idx_bufs[m % 2]
        idd.wait()
        for k in range(MBC):
            if k == NB and m + 1 < MB:
                idd = pltpu.async_copy(col_hbm.at[wid, m + 1],
                                       idx_bufs[(m + 1) % 2],
                                       idx_sems[(m + 1) % 2])
            b = (m * MBC + k) % NB
            if sd[b] is not None:
                sd[b].wait()
            sd[b] = pltpu.async_copy(ones_v, acc_sh.at[cidx_v.at[k]],
                                     sems[b], add=True)
    for d in sd:
        d.wait()
    plsc.subcore_barrier()
    _rows_copy(sid, lambda b, n: pltpu.sync_copy(
        acc_sh.at[pl.ds(b, n)], deg_hbm.at[cid, pl.ds(b, n)]))


@functools.partial(
    pl.kernel, mesh=_mesh,
    out_type=jax.ShapeDtypeStruct((NC, NA, D), jnp.float32),
    scratch_types=[
        pltpu.VMEM_SHARED((NA, D), jnp.float32),
    ] + [pltpu.VMEM((CH, D), jnp.float32)] * NB + [
        pltpu.VMEM((MBC, CH), jnp.int32),
        pltpu.VMEM((MBC, CH), jnp.int32),
        pltpu.VMEM((MBC, CH), jnp.int32),
        pltpu.VMEM((MBC, CH), jnp.int32),
        pltpu.SemaphoreType.DMA,
        pltpu.SemaphoreType.DMA,
    ] + [pltpu.SemaphoreType.DMA] * (2 * NB),
)
def _sc_scatter(y_hbm, row_hbm, col_hbm, zeros_hbm, acc_hbm, acc_sh, *rest):
    rows = rest[:NB]
    idx_bufs = ((rest[NB], rest[NB + 1]), (rest[NB + 2], rest[NB + 3]))
    idx_sems = (rest[NB + 4], rest[NB + 5])
    sem_g = rest[NB + 6:NB + 6 + NB]
    sem_s = rest[NB + 6 + NB:]
    cid = lax.axis_index("c")
    sid = lax.axis_index("s")
    wid = sid * NC + cid
    _rows_copy(sid, lambda b, n: pltpu.sync_copy(
        zeros_hbm.at[pl.ds(b, n)], acc_sh.at[pl.ds(b, n)]))
    plsc.subcore_barrier()

    NCH = MB * MBC

    def issue_idx(m):
        r, c = idx_bufs[m % 2]
        s = idx_sems[m % 2]
        return (pltpu.async_copy(row_hbm.at[wid, m], r, s),
                pltpu.async_copy(col_hbm.at[wid, m], c, s))

    def issue_scatter(j):
        jm, jk = divmod(j, MBC)
        cb = idx_bufs[jm % 2][1]
        return pltpu.async_copy(rows[j % NB], acc_sh.at[cb.at[jk]],
                                sem_s[j % NB], add=True)

    gd = [None] * NCH
    sd = [None] * NCH
    idd = issue_idx(0)
    for kk in range(NCH):
        m, k = divmod(kk, MBC)
        if k == 0:
            for d in idd:
                d.wait()
            ridx_v = idx_bufs[m % 2][0]
        if k == NB and m + 1 < MB:
            idd = issue_idx(m + 1)
        if kk >= NB:
            sd[kk - NB].wait()
        gd[kk] = pltpu.async_copy(y_hbm.at[ridx_v.at[k]], rows[kk % NB],
                                  sem_g[kk % NB])
        if kk >= STG:
            gd[kk - STG].wait()
            sd[kk - STG] = issue_scatter(kk - STG)
    for j in range(NCH - STG, NCH):
        gd[j].wait()
        sd[j] = issue_scatter(j)
    for j in range(NCH - NB, NCH):
        sd[j].wait()
    plsc.subcore_barrier()
    _rows_copy(sid, lambda b, n: pltpu.sync_copy(
        acc_sh.at[pl.ds(b, n)], acc_hbm.at[cid, pl.ds(b, n)]))


def _tc1_body(dp_ref, x_ref, w_ref, y_ref, dinv_ref):
    dp = dp_ref[...]
    deg = dp[0, :, 0] + dp[1, :, 0] + 1.0
    dinv = lax.rsqrt(deg)[:, None]
    xw = jnp.dot(x_ref[...], w_ref[...], preferred_element_type=jnp.float32)
    y_ref[...] = xw * dinv
    dinv_ref[...] = dinv


_tc1 = pl.pallas_call(
    _tc1_body,
    grid=(GRID,),
    in_specs=[
        pl.BlockSpec((NC, BM, D), lambda i: (0, i, 0)),
        pl.BlockSpec((BM, D), lambda i: (i, 0)),
        pl.BlockSpec((D, D), lambda i: (0, 0)),
    ],
    out_specs=[
        pl.BlockSpec((BM, D), lambda i: (i, 0)),
        pl.BlockSpec((BM, 1), lambda i: (i, 0)),
    ],
    out_shape=[
        jax.ShapeDtypeStruct((N, D), jnp.float32),
        jax.ShapeDtypeStruct((N, 1), jnp.float32),
    ],
)


def _tc2_body(dinv_ref, a_ref, y1_ref, b_ref, w_ref, y2_ref):
    dinv = dinv_ref[...]
    a = a_ref[...]
    s = a[0] + a[1] + y1_ref[...]
    h = jnp.maximum(s * dinv + b_ref[...], 0.0)
    hw = jnp.dot(h, w_ref[...], preferred_element_type=jnp.float32)
    y2_ref[...] = hw * dinv


_tc2 = pl.pallas_call(
    _tc2_body,
    grid=(GRID,),
    in_specs=[
        pl.BlockSpec((BM, 1), lambda i: (i, 0)),
        pl.BlockSpec((NC, BM, D), lambda i: (0, i, 0)),
        pl.BlockSpec((BM, D), lambda i: (i, 0)),
        pl.BlockSpec((1, D), lambda i: (0, 0)),
        pl.BlockSpec((D, D), lambda i: (0, 0)),
    ],
    out_specs=pl.BlockSpec((BM, D), lambda i: (i, 0)),
    out_shape=jax.ShapeDtypeStruct((N, D), jnp.float32),
)


def _tc3_body(dinv_ref, a_ref, y2_ref, b_ref, o_ref):
    dinv = dinv_ref[...]
    a = a_ref[...]
    o_ref[...] = (a[0] + a[1] + y2_ref[...]) * dinv + b_ref[...]


_tc3 = pl.pallas_call(
    _tc3_body,
    grid=(GRID,),
    in_specs=[
        pl.BlockSpec((BM, 1), lambda i: (i, 0)),
        pl.BlockSpec((NC, BM, D), lambda i: (0, i, 0)),
        pl.BlockSpec((BM, D), lambda i: (i, 0)),
        pl.BlockSpec((1, D), lambda i: (0, 0)),
    ],
    out_specs=pl.BlockSpec((BM, D), lambda i: (i, 0)),
    out_shape=jax.ShapeDtypeStruct((N, D), jnp.float32),
)


def kernel(x, edge_index, W1, b1, W2, b2):
    row = edge_index[0].reshape(NW, MB, MBC, CH)
    col = edge_index[1].reshape(NW, MB, MBC, CH)
    ones = jnp.ones((CH, D), jnp.float32)
    zeros = jnp.zeros((NA, D), jnp.float32)
    deg = _sc_degree(col, ones, zeros)
    y1, dinv = _tc1(deg, x, W1)
    acc1 = _sc_scatter(y1, row, col, zeros)
    y2 = _tc2(dinv, acc1, y1, b1.reshape(1, D), W2)
    acc2 = _sc_scatter(y2, row, col, zeros)
    out = _tc3(dinv, acc2, y2, b2.reshape(1, D))
    return out
